# Initial kernel scaffold; baseline (speedup 1.0000x reference)
import jax
import jax.numpy as jnp
from jax.experimental import pallas as pl


def kernel(x, edge_index, W1, b1, W2, b2, W3, b3, Wc, bc):
    raise NotImplementedError("write your pallas kernel here")


# speedup vs baseline: 29.4406x; 29.4406x over previous
import functools

import jax
import jax.numpy as jnp
from jax import lax
from jax.experimental import pallas as pl
from jax.experimental.pallas import tpu as pltpu
from jax.experimental.pallas import tpu_sc as plsc

N = 10000
E = 320000
D = 128
C = 8

NC = 2
NS = 16
NW = NC * NS
CHUNK = 125
NCHUNKS = E // CHUNK
CPT = NCHUNKS // NW
RPT = N // NS

_f32 = jnp.float32


def _sc_mesh():
    return plsc.VectorSubcoreMesh(core_axis_name="c", subcore_axis_name="s")


def _make_agg(F):
    @functools.partial(
        pl.kernel,
        out_type=jax.ShapeDtypeStruct((NC, N, F), _f32),
        mesh=_sc_mesh(),
        compiler_params=pltpu.CompilerParams(use_tc_tiling_on_sc=False),
        scratch_types=[
            pltpu.VMEM((CPT, CHUNK), jnp.int32),
            pltpu.VMEM((CPT, CHUNK), jnp.int32),
            pltpu.VMEM((CHUNK, F), _f32),
            pltpu.VMEM_SHARED((N, F), _f32),
            pltpu.SemaphoreType.DMA,
        ],
    )
    def agg(hs_hbm, src_hbm, dst_hbm, zero_hbm, out_hbm,
            src_v, dst_v, msg_v, acc_sh, sem):
        c = lax.axis_index("c")
        s = lax.axis_index("s")
        w = c * NS + s
        pltpu.sync_copy(src_hbm.at[pl.ds(w * CPT, CPT)], src_v)
        pltpu.sync_copy(dst_hbm.at[pl.ds(w * CPT, CPT)], dst_v)

        @pl.when(s == 0)
        def _():
            pltpu.sync_copy(zero_hbm, acc_sh)

        plsc.subcore_barrier()

        def step(j, carry):
            pltpu.async_copy(hs_hbm.at[src_v.at[j]], msg_v, sem).wait()
            pltpu.sync_copy(msg_v, acc_sh.at[dst_v.at[j]], add=True)
            return carry

        lax.fori_loop(0, CPT, step, 0)
        plsc.subcore_barrier()

        @pl.when(s == 0)
        def _():
            pltpu.sync_copy(acc_sh, out_hbm.at[c])

    return agg


_agg4 = _make_agg(4)
_agg2 = _make_agg(2)


@functools.partial(
    pl.kernel,
    out_type=jax.ShapeDtypeStruct((NC, N, 1), _f32),
    mesh=_sc_mesh(),
    compiler_params=pltpu.CompilerParams(use_tc_tiling_on_sc=False),
    scratch_types=[
        pltpu.VMEM((CPT, CHUNK), jnp.int32),
        pltpu.VMEM((CHUNK, 1), _f32),
        pltpu.VMEM_SHARED((N, 1), _f32),
    ],
)
def _deg_kernel(dst_hbm, ones_hbm, zero_hbm, out_hbm, dst_v, ones_v, acc_sh):
    c = lax.axis_index("c")
    s = lax.axis_index("s")
    w = c * NS + s
    pltpu.sync_copy(dst_hbm.at[pl.ds(w * CPT, CPT)], dst_v)
    pltpu.sync_copy(ones_hbm, ones_v)

    @pl.when(s == 0)
    def _():
        pltpu.sync_copy(zero_hbm, acc_sh)

    plsc.subcore_barrier()

    def step(j, carry):
        pltpu.sync_copy(ones_v, acc_sh.at[dst_v.at[j]], add=True)
        return carry

    lax.fori_loop(0, CPT, step, 0)
    plsc.subcore_barrier()

    @pl.when(s == 0)
    def _():
        pltpu.sync_copy(acc_sh, out_hbm.at[c])


_RB = 1000
_GRID = N // _RB


def _small_matmul(t, w_ref, f_in):
    acc = t[:, 0:1] * w_ref[0:1, :]
    for k in range(1, f_in):
        acc = acc + t[:, k:k + 1] * w_ref[k:k + 1, :]
    return acc


def _tc1_body(x_ref, w_ref, deg2_ref, hs_ref, dis_ref):
    deg = deg2_ref[0] + deg2_ref[1] + 1.0
    dis = lax.rsqrt(deg)
    h = jnp.dot(x_ref[...], w_ref[...], preferred_element_type=_f32)
    hs_ref[...] = h * dis
    dis_ref[...] = dis


def _tc1(x, w1, deg2):
    return pl.pallas_call(
        _tc1_body,
        grid=(_GRID,),
        in_specs=[
            pl.BlockSpec((_RB, D), lambda i: (i, 0)),
            pl.BlockSpec((D, 4), lambda i: (0, 0)),
            pl.BlockSpec((NC, _RB, 1), lambda i: (0, i, 0)),
        ],
        out_specs=[
            pl.BlockSpec((_RB, 4), lambda i: (i, 0)),
            pl.BlockSpec((_RB, 1), lambda i: (i, 0)),
        ],
        out_shape=[
            jax.ShapeDtypeStruct((N, 4), _f32),
            jax.ShapeDtypeStruct((N, 1), _f32),
        ],
    )(x, w1, deg2)


def _make_tc2(f_in, f_out):
    def body(agg2_ref, hsp_ref, dis_ref, b_ref, w_ref, hsn_ref):
        dis = dis_ref[...]
        t = jnp.tanh(dis * (agg2_ref[0] + agg2_ref[1] + hsp_ref[...])
                     + b_ref[...])
        hsn_ref[...] = _small_matmul(t, w_ref, f_in) * dis

    def run(agg2, hsp, dis, b, w):
        return pl.pallas_call(
            body,
            grid=(_GRID,),
            in_specs=[
                pl.BlockSpec((NC, _RB, f_in), lambda i: (0, i, 0)),
                pl.BlockSpec((_RB, f_in), lambda i: (i, 0)),
                pl.BlockSpec((_RB, 1), lambda i: (i, 0)),
                pl.BlockSpec((1, f_in), lambda i: (0, 0)),
                pl.BlockSpec((f_in, f_out), lambda i: (0, 0)),
            ],
            out_specs=pl.BlockSpec((_RB, f_out), lambda i: (i, 0)),
            out_shape=jax.ShapeDtypeStruct((N, f_out), _f32),
        )(agg2, hsp, dis, b, w)

    return run


_tc2_44 = _make_tc2(4, 4)
_tc2_42 = _make_tc2(4, 2)


def _tc3_body(agg2_ref, hsp_ref, dis_ref, b_ref, wc_ref, bc_ref,
              out_ref, h_ref):
    dis = dis_ref[...]
    h = jnp.tanh(dis * (agg2_ref[0] + agg2_ref[1] + hsp_ref[...])
                 + b_ref[...])
    h_ref[...] = h
    out_ref[...] = _small_matmul(h, wc_ref, 2) + bc_ref[...]


def _tc3(agg2, hsp, dis, b3, wc, bc):
    return pl.pallas_call(
        _tc3_body,
        grid=(_GRID,),
        in_specs=[
            pl.BlockSpec((NC, _RB, 2), lambda i: (0, i, 0)),
            pl.BlockSpec((_RB, 2), lambda i: (i, 0)),
            pl.BlockSpec((_RB, 1), lambda i: (i, 0)),
            pl.BlockSpec((1, 2), lambda i: (0, 0)),
            pl.BlockSpec((2, C), lambda i: (0, 0)),
            pl.BlockSpec((1, C), lambda i: (0, 0)),
        ],
        out_specs=[
            pl.BlockSpec((_RB, C), lambda i: (i, 0)),
            pl.BlockSpec((_RB, 2), lambda i: (i, 0)),
        ],
        out_shape=[
            jax.ShapeDtypeStruct((N, C), _f32),
            jax.ShapeDtypeStruct((N, 2), _f32),
        ],
    )(agg2, hsp, dis, b3, wc, bc)


def kernel(x, edge_index, W1, b1, W2, b2, W3, b3, Wc, bc):
    ei = edge_index.reshape(2, NCHUNKS, CHUNK)
    src = ei[0]
    dst = ei[1]
    zero4 = jnp.zeros((N, 4), _f32)
    zero2 = jnp.zeros((N, 2), _f32)
    zero1 = jnp.zeros((N, 1), _f32)
    ones = jnp.ones((CHUNK, 1), _f32)

    deg2 = _deg_kernel(dst, ones, zero1)
    hs1, dis = _tc1(x, W1, deg2)
    a1 = _agg4(hs1, src, dst, zero4)
    hs2 = _tc2_44(a1, hs1, dis, b1.reshape(1, 4), W2)
    a2 = _agg4(hs2, src, dst, zero4)
    hs3 = _tc2_42(a2, hs2, dis, b2.reshape(1, 4), W3)
    a3 = _agg2(hs3, src, dst, zero2)
    out, h = _tc3(a3, hs3, dis, b3.reshape(1, 2), Wc, bc.reshape(1, C))
    return (out, h)


# speedup vs baseline: 47.2646x; 1.6054x over previous
import functools

import jax
import jax.numpy as jnp
from jax import lax
from jax.experimental import pallas as pl
from jax.experimental.pallas import tpu as pltpu
from jax.experimental.pallas import tpu_sc as plsc

N = 10000
E = 320000
D = 128
C = 8

NC = 2
NS = 16
NW = NC * NS
CHUNK = 125
NCHUNKS = E // CHUNK
CPT = NCHUNKS // NW
RPT = N // NS

_f32 = jnp.float32


def _sc_mesh():
    return plsc.VectorSubcoreMesh(core_axis_name="c", subcore_axis_name="s")


def _make_agg(F):
    @functools.partial(
        pl.kernel,
        out_type=jax.ShapeDtypeStruct((NC, N, F), _f32),
        mesh=_sc_mesh(),
        compiler_params=pltpu.CompilerParams(use_tc_tiling_on_sc=False),
        scratch_types=[
            pltpu.VMEM((CPT, CHUNK), jnp.int32),
            pltpu.VMEM((CPT, CHUNK), jnp.int32),
            pltpu.VMEM((8, CHUNK, F), _f32),
            pltpu.VMEM_SHARED((N, F), _f32),
            [pltpu.SemaphoreType.DMA] * 8,
        ],
    )
    def agg(hs_hbm, src_hbm, dst_hbm, zero_hbm, out_hbm,
            src_v, dst_v, msg_v, acc_sh, sems):
        c = lax.axis_index("c")
        s = lax.axis_index("s")
        w = c * NS + s
        pltpu.sync_copy(src_hbm.at[pl.ds(w * CPT, CPT)], src_v)
        pltpu.sync_copy(dst_hbm.at[pl.ds(w * CPT, CPT)], dst_v)

        @pl.when(s == 0)
        def _():
            pltpu.sync_copy(zero_hbm, acc_sh)

        plsc.subcore_barrier()

        GDEPTH = 8
        NGROUPS = CPT // GDEPTH

        def group(g, carry):
            base = g * GDEPTH
            copies = []
            for b in range(GDEPTH):
                copies.append(pltpu.async_copy(
                    hs_hbm.at[src_v.at[base + b]],
                    msg_v.at[b], sems[b]))
            for b in range(GDEPTH):
                copies[b].wait()
                pltpu.sync_copy(msg_v.at[b],
                                acc_sh.at[dst_v.at[base + b]], add=True)
            return carry

        lax.fori_loop(0, NGROUPS, group, 0)
        plsc.subcore_barrier()

        @pl.when(s == 0)
        def _():
            pltpu.sync_copy(acc_sh, out_hbm.at[c])

    return agg


_agg4 = _make_agg(4)
_agg2 = _make_agg(2)


@functools.partial(
    pl.kernel,
    out_type=jax.ShapeDtypeStruct((NC, N, 1), _f32),
    mesh=_sc_mesh(),
    compiler_params=pltpu.CompilerParams(use_tc_tiling_on_sc=False),
    scratch_types=[
        pltpu.VMEM((CPT, CHUNK), jnp.int32),
        pltpu.VMEM((CHUNK, 1), _f32),
        pltpu.VMEM_SHARED((N, 1), _f32),
    ],
)
def _deg_kernel(dst_hbm, ones_hbm, zero_hbm, out_hbm, dst_v, ones_v, acc_sh):
    c = lax.axis_index("c")
    s = lax.axis_index("s")
    w = c * NS + s
    pltpu.sync_copy(dst_hbm.at[pl.ds(w * CPT, CPT)], dst_v)
    pltpu.sync_copy(ones_hbm, ones_v)

    @pl.when(s == 0)
    def _():
        pltpu.sync_copy(zero_hbm, acc_sh)

    plsc.subcore_barrier()

    def step(j, carry):
        pltpu.sync_copy(ones_v, acc_sh.at[dst_v.at[j]], add=True)
        return carry

    lax.fori_loop(0, CPT, step, 0)
    plsc.subcore_barrier()

    @pl.when(s == 0)
    def _():
        pltpu.sync_copy(acc_sh, out_hbm.at[c])


_RB = 1000
_GRID = N // _RB


def _small_matmul(t, w_ref, f_in):
    acc = t[:, 0:1] * w_ref[0:1, :]
    for k in range(1, f_in):
        acc = acc + t[:, k:k + 1] * w_ref[k:k + 1, :]
    return acc


def _tc1_body(x_ref, w_ref, deg2_ref, hs_ref, dis_ref):
    deg = deg2_ref[0] + deg2_ref[1] + 1.0
    dis = lax.rsqrt(deg)
    h = jnp.dot(x_ref[...], w_ref[...], preferred_element_type=_f32)
    hs_ref[...] = h * dis
    dis_ref[...] = dis


def _tc1(x, w1, deg2):
    return pl.pallas_call(
        _tc1_body,
        grid=(_GRID,),
        in_specs=[
            pl.BlockSpec((_RB, D), lambda i: (i, 0)),
            pl.BlockSpec((D, 4), lambda i: (0, 0)),
            pl.BlockSpec((NC, _RB, 1), lambda i: (0, i, 0)),
        ],
        out_specs=[
            pl.BlockSpec((_RB, 4), lambda i: (i, 0)),
            pl.BlockSpec((_RB, 1), lambda i: (i, 0)),
        ],
        out_shape=[
            jax.ShapeDtypeStruct((N, 4), _f32),
            jax.ShapeDtypeStruct((N, 1), _f32),
        ],
    )(x, w1, deg2)


def _make_tc2(f_in, f_out):
    def body(agg2_ref, hsp_ref, dis_ref, b_ref, w_ref, hsn_ref):
        dis = dis_ref[...]
        t = jnp.tanh(dis * (agg2_ref[0] + agg2_ref[1] + hsp_ref[...])
                     + b_ref[...])
        hsn_ref[...] = _small_matmul(t, w_ref, f_in) * dis

    def run(agg2, hsp, dis, b, w):
        return pl.pallas_call(
            body,
            grid=(_GRID,),
            in_specs=[
                pl.BlockSpec((NC, _RB, f_in), lambda i: (0, i, 0)),
                pl.BlockSpec((_RB, f_in), lambda i: (i, 0)),
                pl.BlockSpec((_RB, 1), lambda i: (i, 0)),
                pl.BlockSpec((1, f_in), lambda i: (0, 0)),
                pl.BlockSpec((f_in, f_out), lambda i: (0, 0)),
            ],
            out_specs=pl.BlockSpec((_RB, f_out), lambda i: (i, 0)),
            out_shape=jax.ShapeDtypeStruct((N, f_out), _f32),
        )(agg2, hsp, dis, b, w)

    return run


_tc2_44 = _make_tc2(4, 4)
_tc2_42 = _make_tc2(4, 2)


def _tc3_body(agg2_ref, hsp_ref, dis_ref, b_ref, wc_ref, bc_ref,
              out_ref, h_ref):
    dis = dis_ref[...]
    h = jnp.tanh(dis * (agg2_ref[0] + agg2_ref[1] + hsp_ref[...])
                 + b_ref[...])
    h_ref[...] = h
    out_ref[...] = _small_matmul(h, wc_ref, 2) + bc_ref[...]


def _tc3(agg2, hsp, dis, b3, wc, bc):
    return pl.pallas_call(
        _tc3_body,
        grid=(_GRID,),
        in_specs=[
            pl.BlockSpec((NC, _RB, 2), lambda i: (0, i, 0)),
            pl.BlockSpec((_RB, 2), lambda i: (i, 0)),
            pl.BlockSpec((_RB, 1), lambda i: (i, 0)),
            pl.BlockSpec((1, 2), lambda i: (0, 0)),
            pl.BlockSpec((2, C), lambda i: (0, 0)),
            pl.BlockSpec((1, C), lambda i: (0, 0)),
        ],
        out_specs=[
            pl.BlockSpec((_RB, C), lambda i: (i, 0)),
            pl.BlockSpec((_RB, 2), lambda i: (i, 0)),
        ],
        out_shape=[
            jax.ShapeDtypeStruct((N, C), _f32),
            jax.ShapeDtypeStruct((N, 2), _f32),
        ],
    )(agg2, hsp, dis, b3, wc, bc)


def kernel(x, edge_index, W1, b1, W2, b2, W3, b3, Wc, bc):
    ei = edge_index.reshape(2, NCHUNKS, CHUNK)
    src = ei[0]
    dst = ei[1]
    zero4 = jnp.zeros((N, 4), _f32)
    zero2 = jnp.zeros((N, 2), _f32)
    zero1 = jnp.zeros((N, 1), _f32)
    ones = jnp.ones((CHUNK, 1), _f32)

    deg2 = _deg_kernel(dst, ones, zero1)
    hs1, dis = _tc1(x, W1, deg2)
    a1 = _agg4(hs1, src, dst, zero4)
    hs2 = _tc2_44(a1, hs1, dis, b1.reshape(1, 4), W2)
    a2 = _agg4(hs2, src, dst, zero4)
    hs3 = _tc2_42(a2, hs2, dis, b2.reshape(1, 4), W3)
    a3 = _agg2(hs3, src, dst, zero2)
    out, h = _tc3(a3, hs3, dis, b3.reshape(1, 2), Wc, bc.reshape(1, C))
    return (out, h)
